# Initial kernel scaffold; baseline (speedup 1.0000x reference)
import jax
import jax.numpy as jnp
from jax.experimental import pallas as pl


def kernel(x, w0, w1, w2, w3, w4, w5, w_genres, w_title):
    raise NotImplementedError("write your pallas kernel here")


# speedup vs baseline: 4.6230x; 4.6230x over previous
import jax
import jax.numpy as jnp
from jax import lax
from jax.experimental import pallas as pl
from jax.experimental.pallas import tpu as pltpu
from jax.experimental.pallas import tpu_sc as plsc

VOCAB = 100000
EMB = 32
B = 16384

NC = 2
NS = 16
NW = NC * NS
BPW = B // NW
C = 64
NCHUNK = BPW // C
LG = 4
LT = 20
NF = 30


def _body(xT, w0, w1, w2, w3, w4, w5, wg, wt, out,
          idx_v, oh_v, g_v, t_v, pg_v, pt_v,
          invg_v, c0g_v, invt_v, c0t_v, t0g_v, t0t_v, sem):
    onehot = [w0, w1, w2, w3, w4, w5]
    wid = lax.axis_index("s") * NC + lax.axis_index("c")
    base = wid * BPW

    pltpu.sync_copy(wg.at[0], t0g_v)
    pltpu.sync_copy(wt.at[0], t0t_v)

    def chunk(c, carry):
        off = base + c * C
        pltpu.sync_copy(xT.at[:, pl.ds(off, C)], idx_v)

        cps = []
        for f in range(6):
            cps.append(pltpu.make_async_copy(
                onehot[f].at[idx_v.at[f]], oh_v.at[f], sem))
        for l in range(LG):
            cps.append(pltpu.make_async_copy(
                wg.at[idx_v.at[6 + l]], g_v.at[l], sem))
        for l in range(LT):
            cps.append(pltpu.make_async_copy(
                wt.at[idx_v.at[10 + l]], t_v.at[l], sem))
        for cp in cps:
            cp.start()

        for blk in range(C // 16):
            i0 = blk * 16
            cnt_g = jnp.zeros((16,), jnp.float32)
            for l in range(LG):
                v = idx_v[6 + l, pl.ds(i0, 16)]
                cnt_g = cnt_g + jnp.where(v > 0, 1.0, 0.0)
            invg_v[pl.ds(i0, 16)] = 1.0 / cnt_g
            c0g_v[pl.ds(i0, 16)] = float(LG) - cnt_g
            cnt_t = jnp.zeros((16,), jnp.float32)
            for l in range(LT):
                v = idx_v[10 + l, pl.ds(i0, 16)]
                cnt_t = cnt_t + jnp.where(v > 0, 1.0, 0.0)
            invt_v[pl.ds(i0, 16)] = 1.0 / cnt_t
            c0t_v[pl.ds(i0, 16)] = float(LT) - cnt_t

        for cp in cps:
            cp.wait()

        def pool_row(i, _):
            invg = invg_v[pl.ds(i, 16)][0]
            c0g = c0g_v[pl.ds(i, 16)][0]
            invt = invt_v[pl.ds(i, 16)][0]
            c0t = c0t_v[pl.ds(i, 16)][0]
            for dh in range(2):
                d0 = dh * 16
                accg = g_v[0, i, pl.ds(d0, 16)]
                for l in range(1, LG):
                    accg = accg + g_v[l, i, pl.ds(d0, 16)]
                pg_v[i, pl.ds(d0, 16)] = (
                    accg - c0g * t0g_v[pl.ds(d0, 16)]) * invg
                acct = t_v[0, i, pl.ds(d0, 16)]
                for l in range(1, LT):
                    acct = acct + t_v[l, i, pl.ds(d0, 16)]
                pt_v[i, pl.ds(d0, 16)] = (
                    acct - c0t * t0t_v[pl.ds(d0, 16)]) * invt
            return _

        lax.fori_loop(0, C, pool_row, None)

        for f in range(6):
            pltpu.sync_copy(oh_v.at[f],
                            out.at[pl.ds(off, C), pl.ds(f * EMB, EMB)])
        pltpu.sync_copy(pg_v, out.at[pl.ds(off, C), pl.ds(6 * EMB, EMB)])
        pltpu.sync_copy(pt_v, out.at[pl.ds(off, C), pl.ds(7 * EMB, EMB)])
        return carry

    lax.fori_loop(0, NCHUNK, chunk, None)


_sc_call = pl.kernel(
    _body,
    out_type=jax.ShapeDtypeStruct((B, 8 * EMB), jnp.float32),
    mesh=plsc.VectorSubcoreMesh(core_axis_name="c", subcore_axis_name="s"),
    compiler_params=pltpu.CompilerParams(use_tc_tiling_on_sc=False),
    scratch_types=[
        pltpu.VMEM((NF, C), jnp.int32),
        pltpu.VMEM((6, C, EMB), jnp.float32),
        pltpu.VMEM((LG, C, EMB), jnp.float32),
        pltpu.VMEM((LT, C, EMB), jnp.float32),
        pltpu.VMEM((C, EMB), jnp.float32),
        pltpu.VMEM((C, EMB), jnp.float32),
        pltpu.VMEM((C + 16,), jnp.float32),
        pltpu.VMEM((C + 16,), jnp.float32),
        pltpu.VMEM((C + 16,), jnp.float32),
        pltpu.VMEM((C + 16,), jnp.float32),
        pltpu.VMEM((EMB,), jnp.float32),
        pltpu.VMEM((EMB,), jnp.float32),
        pltpu.SemaphoreType.DMA,
    ],
)


def kernel(x, w0, w1, w2, w3, w4, w5, w_genres, w_title):
    xT = x.T
    return _sc_call(xT, w0, w1, w2, w3, w4, w5, w_genres, w_title)
